# baseline (speedup 1.0000x reference)
import functools

import jax
import jax.numpy as jnp
from jax import lax
from jax.experimental import pallas as pl
from jax.experimental.pallas import tpu as pltpu
from jax.experimental.pallas import tpu_sc as plsc

_F = 26
_V1 = 1001
_D = 128
_B = 16384
_R = _B * _F
_NC = 2
_NS = 16
_NW = _NC * _NS
_RPW = _R // _NW
_BPW = _B // _NW
_C = 104
_CB = _C // _F
_NCHUNK = _RPW // _C
_L = 16

_NSLOT = 4
_SC = 2 * _C
_NPAIR = _NCHUNK // 2
_LAGP = 2
_NG = _NPAIR // _NSLOT
_GROUP_SLICES = _NSLOT * _SC // _L

_mesh = plsc.VectorSubcoreMesh(core_axis_name="c", subcore_axis_name="s")


@functools.partial(
    pl.kernel,
    mesh=_mesh,
    out_type=jax.ShapeDtypeStruct((_B, _F * _D), jnp.float32),
    scratch_types=[
        pltpu.VMEM((_RPW,), jnp.int32),
        pltpu.VMEM((_NSLOT * _SC, _D), jnp.float32),
    ] + [pltpu.SemaphoreType.DMA] * (2 * _NSLOT),
)
def _gather_kernel(tab_hbm, lab_hbm, out_hbm, idx_v, big, *sems):
    gsem = sems[:_NSLOT]
    osem = sems[_NSLOT:]

    wid = lax.axis_index("s") * _NC + lax.axis_index("c")
    base = wid * _RPW
    bbase = wid * _BPW
    pltpu.sync_copy(lab_hbm.at[pl.ds(base, _RPW)], idx_v)

    lanes = lax.iota(jnp.int32, _L)

    def compute_idx_group(p):
        for k in range(_GROUP_SLICES):
            off = p * (_NSLOT * _SC) + k * _L
            c = (k * _L) % _F
            v = lanes + c
            field = jnp.where(v >= _F, v - _F, v)
            lv = idx_v[pl.ds(off, _L)]
            idx_v[pl.ds(off, _L)] = (
                field * _V1 + jnp.where(lv == -1, _V1 - 1, lv))

    def gather_copies(k, j):
        lo = pltpu.make_async_copy(
            tab_hbm.at[idx_v.at[pl.ds(2 * k * _C, _C)]],
            big.at[pl.ds(j * _SC, _C)], gsem[j])
        hi = pltpu.make_async_copy(
            tab_hbm.at[idx_v.at[pl.ds((2 * k + 1) * _C, _C)]],
            big.at[pl.ds(j * _SC + _C, _C)], gsem[j])
        return lo, hi

    def out_copy(k, j):
        return pltpu.make_async_copy(
            big.at[pl.ds(j * _SC, _SC)].reshape(2 * _CB, _F * _D),
            out_hbm.at[pl.ds(bbase + k * (2 * _CB), 2 * _CB)], osem[j])

    def start_pair(k, j):
        lo, hi = gather_copies(k, j)
        lo.start()
        hi.start()

    def wait_pair(k, j):
        lo, hi = gather_copies(k, j)
        lo.wait()
        hi.wait()

    compute_idx_group(0)
    for j in range(_NSLOT):
        start_pair(j, j)
        if j >= _LAGP:
            j2 = j - _LAGP
            wait_pair(j2, j2)
            out_copy(j2, j2).start()

    def group_body(p, carry):
        compute_idx_group(p)
        for j in range(_NSLOT):
            k = p * _NSLOT + j
            out_copy(k - _NSLOT, j).wait()
            start_pair(k, j)
            j2 = (j - _LAGP) % _NSLOT
            wait_pair(k - _LAGP, j2)
            out_copy(k - _LAGP, j2).start()
        return carry

    lax.fori_loop(1, _NG, group_body, 0)

    last = _NPAIR - _LAGP
    for i in range(_LAGP):
        k = last + i
        wait_pair(k, k % _NSLOT)
        out_copy(k, k % _NSLOT).start()
    for j in range(_NSLOT):
        out_copy(_NPAIR - _NSLOT + j, j).wait()


def kernel(label, tables):
    lab_flat = label.reshape(_R)
    tab_flat = tables.reshape(_F * _V1, _D)
    return _gather_kernel(tab_flat, lab_flat)


# speedup vs baseline: 1.0016x; 1.0016x over previous
import functools

import jax
import jax.numpy as jnp
from jax import lax
from jax.experimental import pallas as pl
from jax.experimental.pallas import tpu as pltpu
from jax.experimental.pallas import tpu_sc as plsc

_F = 26
_V1 = 1001
_D = 128
_B = 16384
_R = _B * _F
_NC = 2
_NS = 16
_NW = _NC * _NS
_RPW = _R // _NW
_BPW = _B // _NW
_C = 104
_CB = _C // _F
_NCHUNK = _RPW // _C
_L = 16

_NSLOT = 4
_SC = 2 * _C
_NPAIR = _NCHUNK // 2
_LAGP = 3
_NG = _NPAIR // _NSLOT
_GROUP_SLICES = _NSLOT * _SC // _L

_mesh = plsc.VectorSubcoreMesh(core_axis_name="c", subcore_axis_name="s")


@functools.partial(
    pl.kernel,
    mesh=_mesh,
    out_type=jax.ShapeDtypeStruct((_B, _F * _D), jnp.float32),
    scratch_types=[
        pltpu.VMEM((_RPW,), jnp.int32),
        pltpu.VMEM((_NSLOT * _SC, _D), jnp.float32),
    ] + [pltpu.SemaphoreType.DMA] * (2 * _NSLOT),
)
def _gather_kernel(tab_hbm, lab_hbm, out_hbm, idx_v, big, *sems):
    gsem = sems[:_NSLOT]
    osem = sems[_NSLOT:]

    wid = lax.axis_index("s") * _NC + lax.axis_index("c")
    base = wid * _RPW
    bbase = wid * _BPW
    pltpu.sync_copy(lab_hbm.at[pl.ds(base, _RPW)], idx_v)

    lanes = lax.iota(jnp.int32, _L)

    def compute_idx_group(p):
        for k in range(_GROUP_SLICES):
            off = p * (_NSLOT * _SC) + k * _L
            c = (k * _L) % _F
            v = lanes + c
            field = jnp.where(v >= _F, v - _F, v)
            lv = idx_v[pl.ds(off, _L)]
            idx_v[pl.ds(off, _L)] = (
                field * _V1 + jnp.where(lv == -1, _V1 - 1, lv))

    def gather_copies(k, j):
        lo = pltpu.make_async_copy(
            tab_hbm.at[idx_v.at[pl.ds(2 * k * _C, _C)]],
            big.at[pl.ds(j * _SC, _C)], gsem[j])
        hi = pltpu.make_async_copy(
            tab_hbm.at[idx_v.at[pl.ds((2 * k + 1) * _C, _C)]],
            big.at[pl.ds(j * _SC + _C, _C)], gsem[j])
        return lo, hi

    def out_copy(k, j):
        return pltpu.make_async_copy(
            big.at[pl.ds(j * _SC, _SC)].reshape(2 * _CB, _F * _D),
            out_hbm.at[pl.ds(bbase + k * (2 * _CB), 2 * _CB)], osem[j])

    def start_pair(k, j):
        lo, hi = gather_copies(k, j)
        lo.start()
        hi.start()

    def wait_pair(k, j):
        lo, hi = gather_copies(k, j)
        lo.wait()
        hi.wait()

    compute_idx_group(0)
    for j in range(_NSLOT):
        start_pair(j, j)
        if j >= _LAGP:
            j2 = j - _LAGP
            wait_pair(j2, j2)
            out_copy(j2, j2).start()

    def group_body(p, carry):
        compute_idx_group(p)
        for j in range(_NSLOT):
            k = p * _NSLOT + j
            out_copy(k - _NSLOT, j).wait()
            start_pair(k, j)
            j2 = (j - _LAGP) % _NSLOT
            wait_pair(k - _LAGP, j2)
            out_copy(k - _LAGP, j2).start()
        return carry

    lax.fori_loop(1, _NG, group_body, 0)

    last = _NPAIR - _LAGP
    for i in range(_LAGP):
        k = last + i
        wait_pair(k, k % _NSLOT)
        out_copy(k, k % _NSLOT).start()
    for j in range(_NSLOT):
        out_copy(_NPAIR - _NSLOT + j, j).wait()


def kernel(label, tables):
    lab_flat = label.reshape(_R)
    tab_flat = tables.reshape(_F * _V1, _D)
    return _gather_kernel(tab_flat, lab_flat)
